# Initial kernel scaffold; baseline (speedup 1.0000x reference)
import jax
import jax.numpy as jnp
from jax.experimental import pallas as pl


def kernel(inputs, weight):
    raise NotImplementedError("write your pallas kernel here")


# speedup vs baseline: 6.6679x; 6.6679x over previous
import jax
import jax.numpy as jnp
from jax import lax
from jax.experimental import pallas as pl

_N_EMB = 8192
_DIM = 32
_M_BLK = 256


_J_BLK = 4096


def _vq_body(x_ref, a_ref, b_ref, w_ref, q_ref):
    x = x_ref[...]
    w = w_ref[...]
    c = lax.dot_general(x, w, (((1,), (1,)), ((), ())),
                        preferred_element_type=jnp.float32)
    d = (a_ref[...] + b_ref[...]) - 2.0 * c
    acc = jnp.full((d.shape[0], 1), jnp.inf, jnp.float32)
    idx = jnp.zeros((d.shape[0], 1), jnp.int32)
    iota = lax.broadcasted_iota(jnp.int32, (d.shape[0], _J_BLK), 1)
    for g in range(_N_EMB // _J_BLK):
        dg = d[:, g * _J_BLK:(g + 1) * _J_BLK]
        mn = jnp.min(dg, axis=1, keepdims=True)
        ig = jnp.min(jnp.where(dg == mn, iota, _J_BLK),
                     axis=1, keepdims=True) + g * _J_BLK
        win = mn < acc
        acc = jnp.where(win, mn.astype(jnp.bfloat16).astype(jnp.float32), acc)
        idx = jnp.where(win, ig, idx)
    iota_full = lax.broadcasted_iota(jnp.int32, d.shape, 1)
    onehot = (iota_full == idx).astype(jnp.float32)
    q = lax.dot_general(onehot, w, (((1,), (0,)), ((), ())),
                        preferred_element_type=jnp.float32)
    q_ref[...] = x + (q - x)


def kernel(inputs, weight):
    x = jnp.transpose(inputs, (0, 2, 3, 1))
    input_shape = x.shape
    flat = x.reshape(-1, _DIM)
    n = flat.shape[0]
    a = jnp.sum(flat ** 2, axis=1, keepdims=True)
    b = jnp.sum(weight ** 2, axis=1).reshape(1, _N_EMB)

    grid = (n // _M_BLK,)
    q = pl.pallas_call(
        _vq_body,
        grid=grid,
        in_specs=[
            pl.BlockSpec((_M_BLK, _DIM), lambda i: (i, 0)),
            pl.BlockSpec((_M_BLK, 1), lambda i: (i, 0)),
            pl.BlockSpec((1, _N_EMB), lambda i: (0, 0)),
            pl.BlockSpec((_N_EMB, _DIM), lambda i: (0, 0)),
        ],
        out_specs=pl.BlockSpec((_M_BLK, _DIM), lambda i: (i, 0)),
        out_shape=jax.ShapeDtypeStruct((n, _DIM), jnp.float32),
    )(flat, a, b, weight)
    return jnp.transpose(q.reshape(input_shape), (0, 3, 1, 2))


# speedup vs baseline: 10.1685x; 1.5250x over previous
import functools

import jax
import jax.numpy as jnp
from jax import lax
from jax.experimental import pallas as pl
from jax.experimental.pallas import tpu as pltpu
from jax.experimental.pallas import tpu_sc as plsc

_N_EMB = 8192
_DIM = 32
_M_BLK = 256
_J_BLK = 4096

_NC = 2
_NS = 16
_NW = _NC * _NS
_ROWS_PER_W = _N_EMB // _NW
_CHUNK = 128


def _argmin_body(x_ref, a_ref, b_ref, w_ref, idx_ref):
    x = x_ref[...]
    w = w_ref[...]
    c = lax.dot_general(x, w, (((1,), (1,)), ((), ())),
                        preferred_element_type=jnp.float32)
    d = (a_ref[...] + b_ref[...]) - 2.0 * c
    acc = jnp.full((d.shape[0], 1), jnp.inf, jnp.float32)
    idx = jnp.zeros((d.shape[0], 1), jnp.int32)
    iota = lax.broadcasted_iota(jnp.int32, (d.shape[0], _J_BLK), 1)
    for g in range(_N_EMB // _J_BLK):
        dg = d[:, g * _J_BLK:(g + 1) * _J_BLK]
        mn = jnp.min(dg, axis=1, keepdims=True)
        ig = jnp.min(jnp.where(dg == mn, iota, _J_BLK),
                     axis=1, keepdims=True) + g * _J_BLK
        win = mn < acc
        acc = jnp.where(win, mn.astype(jnp.bfloat16).astype(jnp.float32), acc)
        idx = jnp.where(win, ig, idx)
    idx_ref[0, 0, :] = idx[:, 0]


def _nearest_indices(flat, a, b, weight):
    g = flat.shape[0] // _M_BLK
    idx = pl.pallas_call(
        _argmin_body,
        grid=(g,),
        in_specs=[
            pl.BlockSpec((_M_BLK, _DIM), lambda i: (i, 0)),
            pl.BlockSpec((_M_BLK, 1), lambda i: (i, 0)),
            pl.BlockSpec((1, _N_EMB), lambda i: (0, 0)),
            pl.BlockSpec((_N_EMB, _DIM), lambda i: (0, 0)),
        ],
        out_specs=pl.BlockSpec((1, 1, _M_BLK), lambda i: (i, 0, 0)),
        out_shape=jax.ShapeDtypeStruct((g, 1, _M_BLK), jnp.int32),
    )(flat, a, b, weight)
    return idx.reshape(-1)


@functools.partial(
    pl.kernel,
    mesh=plsc.VectorSubcoreMesh(core_axis_name="c", subcore_axis_name="s"),
    compiler_params=pltpu.CompilerParams(use_tc_tiling_on_sc=False),
    out_type=jax.ShapeDtypeStruct((_N_EMB, _DIM), jnp.float32),
    scratch_types=[
        pltpu.VMEM((2, _CHUNK), jnp.int32),
        pltpu.VMEM((_ROWS_PER_W, _DIM), jnp.float32),
        pltpu.SemaphoreType.DMA,
    ],
)
def _sc_gather(w_hbm, idx_hbm, out_hbm, idx_v, rows_v, sem):
    wid = lax.axis_index("s") * _NC + lax.axis_index("c")
    base = wid * _ROWS_PER_W
    for j in range(_ROWS_PER_W // _CHUNK):
        pltpu.sync_copy(idx_hbm.at[pl.ds(base + j * _CHUNK, _CHUNK)],
                        idx_v.at[j])
    copies = [
        pltpu.async_copy(w_hbm.at[idx_v.at[j]],
                         rows_v.at[pl.ds(j * _CHUNK, _CHUNK)], sem)
        for j in range(_ROWS_PER_W // _CHUNK)
    ]
    for cp in copies:
        cp.wait()
    pltpu.sync_copy(rows_v, out_hbm.at[pl.ds(base, _ROWS_PER_W)])


def kernel(inputs, weight):
    x = jnp.transpose(inputs, (0, 2, 3, 1))
    input_shape = x.shape
    flat = x.reshape(-1, _DIM)
    a = jnp.sum(flat ** 2, axis=1, keepdims=True)
    b = jnp.sum(weight ** 2, axis=1).reshape(1, _N_EMB)

    idx = _nearest_indices(flat, a, b, weight)
    wq = weight.astype(jnp.bfloat16).astype(jnp.float32)
    q = _sc_gather(wq, idx)
    return jnp.transpose(q.reshape(input_shape), (0, 3, 1, 2))


# speedup vs baseline: 10.4323x; 1.0259x over previous
import functools

import jax
import jax.numpy as jnp
from jax import lax
from jax.experimental import pallas as pl
from jax.experimental.pallas import tpu as pltpu
from jax.experimental.pallas import tpu_sc as plsc

_N_EMB = 8192
_DIM = 32
_M_BLK = 512
_J_BLK = 4096

_NC = 2
_NS = 16
_NW = _NC * _NS
_ROWS_PER_W = _N_EMB // _NW
_CHUNK = 128


def _argmin_body(x_ref, a_ref, b_ref, w_ref, idx_ref):
    x = x_ref[...]
    w = w_ref[...]
    c = lax.dot_general(x, w, (((1,), (1,)), ((), ())),
                        preferred_element_type=jnp.float32)
    d = (a_ref[...] + b_ref[...]) - 2.0 * c
    acc = jnp.full((d.shape[0], 1), jnp.inf, jnp.float32)
    idx = jnp.zeros((d.shape[0], 1), jnp.int32)
    iota = lax.broadcasted_iota(jnp.int32, (d.shape[0], _J_BLK), 1)
    for g in range(_N_EMB // _J_BLK):
        dg = d[:, g * _J_BLK:(g + 1) * _J_BLK]
        mn = jnp.min(dg, axis=1, keepdims=True)
        ig = jnp.min(jnp.where(dg == mn, iota, _J_BLK),
                     axis=1, keepdims=True) + g * _J_BLK
        win = mn < acc
        acc = jnp.where(win, mn.astype(jnp.bfloat16).astype(jnp.float32), acc)
        idx = jnp.where(win, ig, idx)
    idx_ref[0, 0, :] = idx[:, 0]


def _nearest_indices(flat, a, b, weight):
    g = flat.shape[0] // _M_BLK
    idx = pl.pallas_call(
        _argmin_body,
        grid=(g,),
        in_specs=[
            pl.BlockSpec((_M_BLK, _DIM), lambda i: (i, 0)),
            pl.BlockSpec((_M_BLK, 1), lambda i: (i, 0)),
            pl.BlockSpec((1, _N_EMB), lambda i: (0, 0)),
            pl.BlockSpec((_N_EMB, _DIM), lambda i: (0, 0)),
        ],
        out_specs=pl.BlockSpec((1, 1, _M_BLK), lambda i: (i, 0, 0)),
        out_shape=jax.ShapeDtypeStruct((g, 1, _M_BLK), jnp.int32),
    )(flat, a, b, weight)
    return idx.reshape(-1)


@functools.partial(
    pl.kernel,
    mesh=plsc.VectorSubcoreMesh(core_axis_name="c", subcore_axis_name="s"),
    compiler_params=pltpu.CompilerParams(use_tc_tiling_on_sc=False),
    out_type=jax.ShapeDtypeStruct((_N_EMB, _DIM), jnp.float32),
    scratch_types=[
        pltpu.VMEM((2, _CHUNK), jnp.int32),
        pltpu.VMEM((_ROWS_PER_W, _DIM), jnp.float32),
        pltpu.SemaphoreType.DMA,
    ],
)
def _sc_gather(w_hbm, idx_hbm, out_hbm, idx_v, rows_v, sem):
    wid = lax.axis_index("s") * _NC + lax.axis_index("c")
    base = wid * _ROWS_PER_W
    for j in range(_ROWS_PER_W // _CHUNK):
        pltpu.sync_copy(idx_hbm.at[pl.ds(base + j * _CHUNK, _CHUNK)],
                        idx_v.at[j])
    copies = [
        pltpu.async_copy(w_hbm.at[idx_v.at[j]],
                         rows_v.at[pl.ds(j * _CHUNK, _CHUNK)], sem)
        for j in range(_ROWS_PER_W // _CHUNK)
    ]
    for cp in copies:
        cp.wait()
    pltpu.sync_copy(rows_v, out_hbm.at[pl.ds(base, _ROWS_PER_W)])


def kernel(inputs, weight):
    x = jnp.transpose(inputs, (0, 2, 3, 1))
    input_shape = x.shape
    flat = x.reshape(-1, _DIM)
    a = jnp.sum(flat ** 2, axis=1, keepdims=True)
    b = jnp.sum(weight ** 2, axis=1).reshape(1, _N_EMB)

    idx = _nearest_indices(flat, a, b, weight)
    wq = weight.astype(jnp.bfloat16).astype(jnp.float32)
    q = _sc_gather(wq, idx)
    return jnp.transpose(q.reshape(input_shape), (0, 3, 1, 2))


# speedup vs baseline: 11.8689x; 1.1377x over previous
import functools

import jax
import jax.numpy as jnp
from jax import lax
from jax.experimental import pallas as pl
from jax.experimental.pallas import tpu as pltpu
from jax.experimental.pallas import tpu_sc as plsc

_N_EMB = 8192
_DIM = 32
_M_BLK = 512
_J_BLK = 4096

_NC = 2
_NS = 16
_NW = _NC * _NS
_ROWS_PER_W = _N_EMB // _NW
_CHUNK = 128


_LANES = 128


def _argmin_body(x_ref, a_ref, b_ref, w_ref, idx_ref):
    x = x_ref[...]
    w = w_ref[...]
    c = lax.dot_general(x, w, (((1,), (1,)), ((), ())),
                        preferred_element_type=jnp.float32)
    m = x.shape[0]
    a = a_ref[...]
    b = b_ref[...]
    lane = lax.broadcasted_iota(jnp.int32, (m, _LANES), 1).astype(jnp.float32)
    acc = jnp.full((m, 1), jnp.inf, jnp.float32)
    idx = jnp.zeros((m, 1), jnp.float32)
    for g in range(_N_EMB // _J_BLK):
        accv = jnp.full((m, _LANES), jnp.inf, jnp.float32)
        vsel = jnp.zeros((m, _LANES), jnp.float32)
        for v in range(_J_BLK // _LANES):
            lo = g * _J_BLK + v * _LANES
            dv = (a + b[:, lo:lo + _LANES]) - 2.0 * c[:, lo:lo + _LANES]
            win = dv < accv
            accv = jnp.where(win, dv, accv)
            vsel = jnp.where(win, jnp.float32(v), vsel)
        mn = jnp.min(accv, axis=1, keepdims=True)
        j = vsel * jnp.float32(_LANES) + lane + jnp.float32(g * _J_BLK)
        ig = jnp.min(jnp.where(accv == mn, j, jnp.float32(_N_EMB)),
                     axis=1, keepdims=True)
        win = mn < acc
        acc = jnp.where(win, mn.astype(jnp.bfloat16).astype(jnp.float32), acc)
        idx = jnp.where(win, ig, idx)
    idx_ref[0, 0, :] = idx[:, 0].astype(jnp.int32)


def _nearest_indices(flat, a, b, weight):
    g = flat.shape[0] // _M_BLK
    idx = pl.pallas_call(
        _argmin_body,
        grid=(g,),
        in_specs=[
            pl.BlockSpec((_M_BLK, _DIM), lambda i: (i, 0)),
            pl.BlockSpec((_M_BLK, 1), lambda i: (i, 0)),
            pl.BlockSpec((1, _N_EMB), lambda i: (0, 0)),
            pl.BlockSpec((_N_EMB, _DIM), lambda i: (0, 0)),
        ],
        out_specs=pl.BlockSpec((1, 1, _M_BLK), lambda i: (i, 0, 0)),
        out_shape=jax.ShapeDtypeStruct((g, 1, _M_BLK), jnp.int32),
    )(flat, a, b, weight)
    return idx.reshape(-1)


@functools.partial(
    pl.kernel,
    mesh=plsc.VectorSubcoreMesh(core_axis_name="c", subcore_axis_name="s"),
    compiler_params=pltpu.CompilerParams(use_tc_tiling_on_sc=False),
    out_type=jax.ShapeDtypeStruct((_N_EMB, _DIM), jnp.float32),
    scratch_types=[
        pltpu.VMEM((2, _CHUNK), jnp.int32),
        pltpu.VMEM((_ROWS_PER_W, _DIM), jnp.float32),
        pltpu.SemaphoreType.DMA,
    ],
)
def _sc_gather(w_hbm, idx_hbm, out_hbm, idx_v, rows_v, sem):
    wid = lax.axis_index("s") * _NC + lax.axis_index("c")
    base = wid * _ROWS_PER_W
    for j in range(_ROWS_PER_W // _CHUNK):
        pltpu.sync_copy(idx_hbm.at[pl.ds(base + j * _CHUNK, _CHUNK)],
                        idx_v.at[j])
    copies = [
        pltpu.async_copy(w_hbm.at[idx_v.at[j]],
                         rows_v.at[pl.ds(j * _CHUNK, _CHUNK)], sem)
        for j in range(_ROWS_PER_W // _CHUNK)
    ]
    for cp in copies:
        cp.wait()
    pltpu.sync_copy(rows_v, out_hbm.at[pl.ds(base, _ROWS_PER_W)])


def kernel(inputs, weight):
    x = jnp.transpose(inputs, (0, 2, 3, 1))
    input_shape = x.shape
    flat = x.reshape(-1, _DIM)
    a = jnp.sum(flat ** 2, axis=1, keepdims=True)
    b = jnp.sum(weight ** 2, axis=1).reshape(1, _N_EMB)

    idx = _nearest_indices(flat, a, b, weight)
    wq = weight.astype(jnp.bfloat16).astype(jnp.float32)
    q = _sc_gather(wq, idx)
    return jnp.transpose(q.reshape(input_shape), (0, 3, 1, 2))


# speedup vs baseline: 11.9808x; 1.0094x over previous
import functools

import jax
import jax.numpy as jnp
from jax import lax
from jax.experimental import pallas as pl
from jax.experimental.pallas import tpu as pltpu
from jax.experimental.pallas import tpu_sc as plsc

_N_EMB = 8192
_DIM = 32
_M_BLK = 1024
_J_BLK = 4096

_NC = 2
_NS = 16
_NW = _NC * _NS
_ROWS_PER_W = _N_EMB // _NW
_CHUNK = 128


_LANES = 128


def _argmin_body(x_ref, a_ref, b_ref, w_ref, idx_ref):
    x = x_ref[...]
    w = w_ref[...]
    c = lax.dot_general(x, w, (((1,), (1,)), ((), ())),
                        preferred_element_type=jnp.float32)
    m = x.shape[0]
    a = a_ref[...]
    b = b_ref[...]
    lane = lax.broadcasted_iota(jnp.int32, (m, _LANES), 1).astype(jnp.float32)
    acc = jnp.full((m, 1), jnp.inf, jnp.float32)
    idx = jnp.zeros((m, 1), jnp.float32)
    for g in range(_N_EMB // _J_BLK):
        accv = jnp.full((m, _LANES), jnp.inf, jnp.float32)
        vsel = jnp.zeros((m, _LANES), jnp.float32)
        for v in range(_J_BLK // _LANES):
            lo = g * _J_BLK + v * _LANES
            dv = (a + b[:, lo:lo + _LANES]) - 2.0 * c[:, lo:lo + _LANES]
            win = dv < accv
            accv = jnp.where(win, dv, accv)
            vsel = jnp.where(win, jnp.float32(v), vsel)
        mn = jnp.min(accv, axis=1, keepdims=True)
        j = vsel * jnp.float32(_LANES) + lane + jnp.float32(g * _J_BLK)
        ig = jnp.min(jnp.where(accv == mn, j, jnp.float32(_N_EMB)),
                     axis=1, keepdims=True)
        win = mn < acc
        acc = jnp.where(win, mn.astype(jnp.bfloat16).astype(jnp.float32), acc)
        idx = jnp.where(win, ig, idx)
    idx_ref[0, 0, :] = idx[:, 0].astype(jnp.int32)


def _nearest_indices(flat, a, b, weight):
    g = flat.shape[0] // _M_BLK
    idx = pl.pallas_call(
        _argmin_body,
        grid=(g,),
        in_specs=[
            pl.BlockSpec((_M_BLK, _DIM), lambda i: (i, 0)),
            pl.BlockSpec((_M_BLK, 1), lambda i: (i, 0)),
            pl.BlockSpec((1, _N_EMB), lambda i: (0, 0)),
            pl.BlockSpec((_N_EMB, _DIM), lambda i: (0, 0)),
        ],
        out_specs=pl.BlockSpec((1, 1, _M_BLK), lambda i: (i, 0, 0)),
        out_shape=jax.ShapeDtypeStruct((g, 1, _M_BLK), jnp.int32),
    )(flat, a, b, weight)
    return idx.reshape(-1)


@functools.partial(
    pl.kernel,
    mesh=plsc.VectorSubcoreMesh(core_axis_name="c", subcore_axis_name="s"),
    compiler_params=pltpu.CompilerParams(use_tc_tiling_on_sc=False),
    out_type=jax.ShapeDtypeStruct((_N_EMB, _DIM), jnp.float32),
    scratch_types=[
        pltpu.VMEM((2, _CHUNK), jnp.int32),
        pltpu.VMEM((_ROWS_PER_W, _DIM), jnp.float32),
        pltpu.SemaphoreType.DMA,
    ],
)
def _sc_gather(w_hbm, idx_hbm, out_hbm, idx_v, rows_v, sem):
    wid = lax.axis_index("s") * _NC + lax.axis_index("c")
    base = wid * _ROWS_PER_W
    for j in range(_ROWS_PER_W // _CHUNK):
        pltpu.sync_copy(idx_hbm.at[pl.ds(base + j * _CHUNK, _CHUNK)],
                        idx_v.at[j])
    copies = [
        pltpu.async_copy(w_hbm.at[idx_v.at[j]],
                         rows_v.at[pl.ds(j * _CHUNK, _CHUNK)], sem)
        for j in range(_ROWS_PER_W // _CHUNK)
    ]
    for cp in copies:
        cp.wait()
    pltpu.sync_copy(rows_v, out_hbm.at[pl.ds(base, _ROWS_PER_W)])


def kernel(inputs, weight):
    x = jnp.transpose(inputs, (0, 2, 3, 1))
    input_shape = x.shape
    flat = x.reshape(-1, _DIM)
    a = jnp.sum(flat ** 2, axis=1, keepdims=True)
    b = jnp.sum(weight ** 2, axis=1).reshape(1, _N_EMB)

    idx = _nearest_indices(flat, a, b, weight)
    wq = weight.astype(jnp.bfloat16).astype(jnp.float32)
    q = _sc_gather(wq, idx)
    return jnp.transpose(q.reshape(input_shape), (0, 3, 1, 2))
